# Initial kernel scaffold; baseline (speedup 1.0000x reference)
import jax
import jax.numpy as jnp
from jax.experimental import pallas as pl


def kernel(rois, roi_cls_loc, roi_scores):
    raise NotImplementedError("write your pallas kernel here")


# speedup vs baseline: 89.1080x; 89.1080x over previous
import functools

import jax
import jax.numpy as jnp
from jax import lax
from jax.experimental import pallas as pl
from jax.experimental.pallas import tpu as pltpu
from jax.experimental.pallas import tpu_sc as plsc

NBOX = 5000
NPAD = 5008
NOUT = 20
NB = NPAD // 16
CPAD = NPAD + 16
NEG = -3.0e38
SCORE_T = 0.05
IOU_T = 0.3
IMG_H = 600.0
IMG_W = 800.0


def _prep_body(rois_ref, loc_ref, sc_ref, out_ref):
    y1r = rois_ref[0:1, :]
    x1r = rois_ref[1:2, :]
    y2r = rois_ref[2:3, :]
    x2r = rois_ref[3:4, :]
    src_h = y2r - y1r
    src_w = x2r - x1r
    src_cy = y1r + 0.5 * src_h
    src_cx = x1r + 0.5 * src_w
    dy = loc_ref[0, 0:1, :] * 0.1 + 0.0
    dx = loc_ref[0, 1:2, :] * 0.1 + 0.0
    dh = loc_ref[0, 2:3, :] * 0.2 + 0.0
    dw = loc_ref[0, 3:4, :] * 0.2 + 0.2
    cy = dy * src_h + src_cy
    cx = dx * src_w + src_cx
    hh = jnp.exp(dh) * src_h
    ww = jnp.exp(dw) * src_w
    yy1 = jnp.clip(cy - 0.5 * hh, 0.0, IMG_H)
    xx1 = jnp.clip(cx - 0.5 * ww, 0.0, IMG_W)
    yy2 = jnp.clip(cy + 0.5 * hh, 0.0, IMG_H)
    xx2 = jnp.clip(cx + 0.5 * ww, 0.0, IMG_W)
    sc = sc_ref[...]
    m = jnp.max(sc, axis=0, keepdims=True)
    e = jnp.exp(sc - m)
    denom = jnp.sum(e, axis=0, keepdims=True)
    probs = e / denom
    g = pl.program_id(0)
    onehot = (lax.broadcasted_iota(jnp.int32, (24, 1), 0) == (g + 1)).astype(
        jnp.float32)
    prob_l = jnp.sum(probs * onehot, axis=0, keepdims=True)
    lanes = lax.broadcasted_iota(jnp.int32, (1, NPAD), 1)
    s_m = jnp.where((prob_l > SCORE_T) & (lanes < NBOX), prob_l, NEG)
    z = jnp.zeros((3, NPAD), jnp.float32)
    out_ref[0] = jnp.concatenate([s_m, yy1, xx1, yy2, xx2, z], axis=0)


def _prep(rois_t, loc_t, sc_t):
    return pl.pallas_call(
        _prep_body,
        grid=(NOUT,),
        in_specs=[
            pl.BlockSpec((8, NPAD), lambda g: (0, 0)),
            pl.BlockSpec((1, 4, NPAD), lambda g: (g + 1, 0, 0)),
            pl.BlockSpec((24, NPAD), lambda g: (0, 0)),
        ],
        out_specs=pl.BlockSpec((1, 8, NPAD), lambda g: (g, 0, 0)),
        out_shape=jax.ShapeDtypeStruct((NOUT, 8, NPAD), jnp.float32),
    )(rois_t, loc_t, sc_t)


def _sc_nms(packed):
    mesh = plsc.VectorSubcoreMesh(
        core_axis_name="c", subcore_axis_name="s", num_cores=2,
        num_subcores=16)

    @functools.partial(
        pl.kernel,
        mesh=mesh,
        out_type=jax.ShapeDtypeStruct((NOUT * 5 * NPAD,), jnp.float32),
        compiler_params=pltpu.CompilerParams(needs_layout_passes=False),
        scratch_types=[pltpu.VMEM((CPAD,), jnp.float32)] * 5
        + [pltpu.VMEM((CPAD,), jnp.float32)] * 6
        + [pltpu.VMEM((CPAD,), jnp.int32)] * 2,
    )
    def k(in_hbm, out_hbm, s_v, y1_v, x1_v, y2_v, x2_v,
          cs, cy1, cx1, cy2, cx2, car, cidx, ci2):
        wid = lax.axis_index("s") * 2 + lax.axis_index("c")

        @pl.when(wid < NOUT)
        def _():
            c = wid
            ibase = c * 8 * NPAD
            pltpu.sync_copy(in_hbm.at[pl.ds(ibase + 0 * NPAD, NPAD)],
                            s_v.at[pl.ds(0, NPAD)])
            pltpu.sync_copy(in_hbm.at[pl.ds(ibase + 1 * NPAD, NPAD)],
                            y1_v.at[pl.ds(0, NPAD)])
            pltpu.sync_copy(in_hbm.at[pl.ds(ibase + 2 * NPAD, NPAD)],
                            x1_v.at[pl.ds(0, NPAD)])
            pltpu.sync_copy(in_hbm.at[pl.ds(ibase + 3 * NPAD, NPAD)],
                            y2_v.at[pl.ds(0, NPAD)])
            pltpu.sync_copy(in_hbm.at[pl.ds(ibase + 4 * NPAD, NPAD)],
                            x2_v.at[pl.ds(0, NPAD)])

            zero16 = jnp.zeros((16,), jnp.float32)
            negs16 = jnp.full((16,), NEG, jnp.float32)
            lane = lax.iota(jnp.int32, 16)
            mask0 = lane == 0
            lane0z = jnp.zeros((16,), jnp.int32)

            def cbody(i, off):
                sl = pl.ds(i * 16, 16)
                sv = s_v[sl]
                msk = sv > 0.0
                a = y1_v[sl]
                b = x1_v[sl]
                cc = y2_v[sl]
                dd = x2_v[sl]
                ar = jnp.maximum(cc - a, 0.0) * jnp.maximum(dd - b, 0.0)
                pos = plsc.cumsum(msk.astype(jnp.int32))
                dst = off + pos - 1
                plsc.store_scatter(cs, [dst], sv, mask=msk)
                plsc.store_scatter(cy1, [dst], a, mask=msk)
                plsc.store_scatter(cx1, [dst], b, mask=msk)
                plsc.store_scatter(cy2, [dst], cc, mask=msk)
                plsc.store_scatter(cx2, [dst], dd, mask=msk)
                plsc.store_scatter(car, [dst], ar, mask=msk)
                plsc.store_scatter(cidx, [dst], i * 16 + lane, mask=msk)
                return off + pos[15]

            moff = lax.fori_loop(0, NB, cbody, jnp.int32(0))
            cs[pl.ds(moff, 16)] = negs16
            cidx[pl.ds(moff, 16)] = lane0z + NPAD
            nbc = (moff + 15) // 16

            def zbody(i, carry):
                sl = pl.ds(i * 16, 16)
                ci2[sl] = i * 16 + lane
                s_v[sl] = zero16
                y1_v[sl] = zero16
                x1_v[sl] = zero16
                y2_v[sl] = zero16
                x2_v[sl] = zero16
                return carry

            lax.fori_loop(0, NB + 1, zbody, 0)

            def am_body(i, carv):
                rm, ri = carv
                sv = cs[pl.ds(i * 16, 16)]
                upd = sv > rm
                return (jnp.where(upd, sv, rm),
                        jnp.where(upd, i * 16 + lane, ri))

            rm0, ri0 = lax.fori_loop(
                0, nbc, am_body, (negs16, jnp.zeros((16,), jnp.int32)))
            g0 = jnp.max(rm0)
            i0 = jnp.min(jnp.where(rm0 == g0, ri0, jnp.int32(1 << 30)))
            i0 = jnp.where(g0 > 0.0, i0, jnp.int32(1 << 30))

            def obody(t, carv):
                gm, gi = carv
                alive = gm > 0.0
                gic = jnp.minimum(gi, moff)
                sel = pl.ds(gic, 16)

                def spl(ref):
                    return ref[sel].at[lane0z].get(mode="promise_in_bounds")

                svalv = spl(cs)
                by1v = spl(cy1)
                bx1v = spl(cx1)
                by2v = spl(cy2)
                bx2v = spl(cx2)
                bareav = spl(car)
                oidxv = cidx[sel]
                plsc.store_scatter(s_v, [oidxv], svalv, mask=mask0)
                plsc.store_scatter(y1_v, [oidxv], by1v, mask=mask0)
                plsc.store_scatter(x1_v, [oidxv], bx1v, mask=mask0)
                plsc.store_scatter(y2_v, [oidxv], by2v, mask=mask0)
                plsc.store_scatter(x2_v, [oidxv], bx2v, mask=mask0)
                plsc.store_scatter(cs, [ci2[sel]], negs16, mask=mask0)
                nbi = jnp.where(alive, nbc, 0)

                def sup(i, carv2):
                    rm, ri = carv2
                    sl = pl.ds(i * 16, 16)
                    sv = cs[sl]
                    tly = jnp.maximum(by1v, cy1[sl])
                    tlx = jnp.maximum(bx1v, cx1[sl])
                    bry = jnp.minimum(by2v, cy2[sl])
                    brx = jnp.minimum(bx2v, cx2[sl])
                    hh = jnp.maximum(bry - tly, 0.0)
                    ww = jnp.maximum(brx - tlx, 0.0)
                    inter = hh * ww
                    iou = inter / (bareav + car[sl] - inter + 1e-8)
                    snew = jnp.where(iou > IOU_T, NEG, sv)
                    cs[sl] = snew
                    upd = snew > rm
                    return (jnp.where(upd, snew, rm),
                            jnp.where(upd, i * 16 + lane, ri))

                rm, ri = lax.fori_loop(
                    0, nbi, sup, (negs16, jnp.zeros((16,), jnp.int32)))
                gm2 = jnp.max(rm)
                gi2 = jnp.min(jnp.where(rm == gm2, ri, jnp.int32(1 << 30)))
                gi2 = jnp.where(gm2 > 0.0, gi2, jnp.int32(1 << 30))
                gm2 = jnp.where(alive, gm2, gm)
                gi2 = jnp.where(alive, gi2, gi)
                return gm2, gi2

            lax.fori_loop(0, moff + 1, obody, (g0, i0))

            obase = c * 5 * NPAD
            pltpu.sync_copy(y1_v.at[pl.ds(0, NPAD)],
                            out_hbm.at[pl.ds(obase + 0 * NPAD, NPAD)])
            pltpu.sync_copy(x1_v.at[pl.ds(0, NPAD)],
                            out_hbm.at[pl.ds(obase + 1 * NPAD, NPAD)])
            pltpu.sync_copy(y2_v.at[pl.ds(0, NPAD)],
                            out_hbm.at[pl.ds(obase + 2 * NPAD, NPAD)])
            pltpu.sync_copy(x2_v.at[pl.ds(0, NPAD)],
                            out_hbm.at[pl.ds(obase + 3 * NPAD, NPAD)])
            pltpu.sync_copy(s_v.at[pl.ds(0, NPAD)],
                            out_hbm.at[pl.ds(obase + 4 * NPAD, NPAD)])

    return k(packed)


def kernel(rois, roi_cls_loc, roi_scores):
    rois_t = jnp.zeros((8, NPAD), jnp.float32).at[:4, :NBOX].set(
        rois.astype(jnp.float32).T)
    loc_t = jnp.zeros((21, 4, NPAD), jnp.float32).at[:, :, :NBOX].set(
        roi_cls_loc.astype(jnp.float32).T.reshape(21, 4, NBOX))
    sc_t = jnp.full((24, NPAD), -1e30, jnp.float32).at[:21, :NBOX].set(
        roi_scores.astype(jnp.float32).T)
    packed = _prep(rois_t, loc_t, sc_t)
    out5 = _sc_nms(packed.reshape(-1)).reshape(NOUT, 5, NPAD)
    return out5.transpose(0, 2, 1)[:, :NBOX, :]


# speedup vs baseline: 161.8095x; 1.8159x over previous
import functools

import jax
import jax.numpy as jnp
from jax import lax
from jax.experimental import pallas as pl
from jax.experimental.pallas import tpu as pltpu
from jax.experimental.pallas import tpu_sc as plsc

NBOX = 5000
NPAD = 5008
NOUT = 20
NB = NPAD // 16
CPAD = NPAD + 16
NEG = -3.0e38
SCORE_T = 0.05
IOU_T = 0.3
IMG_H = 600.0
IMG_W = 800.0


def _prep_body(rois_ref, loc_ref, sc_ref, out_ref):
    y1r = rois_ref[0:1, :]
    x1r = rois_ref[1:2, :]
    y2r = rois_ref[2:3, :]
    x2r = rois_ref[3:4, :]
    src_h = y2r - y1r
    src_w = x2r - x1r
    src_cy = y1r + 0.5 * src_h
    src_cx = x1r + 0.5 * src_w
    dy = loc_ref[0, 0:1, :] * 0.1 + 0.0
    dx = loc_ref[0, 1:2, :] * 0.1 + 0.0
    dh = loc_ref[0, 2:3, :] * 0.2 + 0.0
    dw = loc_ref[0, 3:4, :] * 0.2 + 0.2
    cy = dy * src_h + src_cy
    cx = dx * src_w + src_cx
    hh = jnp.exp(dh) * src_h
    ww = jnp.exp(dw) * src_w
    yy1 = jnp.clip(cy - 0.5 * hh, 0.0, IMG_H)
    xx1 = jnp.clip(cx - 0.5 * ww, 0.0, IMG_W)
    yy2 = jnp.clip(cy + 0.5 * hh, 0.0, IMG_H)
    xx2 = jnp.clip(cx + 0.5 * ww, 0.0, IMG_W)
    sc = sc_ref[...]
    m = jnp.max(sc, axis=0, keepdims=True)
    e = jnp.exp(sc - m)
    denom = jnp.sum(e, axis=0, keepdims=True)
    probs = e / denom
    g = pl.program_id(0)
    onehot = (lax.broadcasted_iota(jnp.int32, (24, 1), 0) == (g + 1)).astype(
        jnp.float32)
    prob_l = jnp.sum(probs * onehot, axis=0, keepdims=True)
    lanes = lax.broadcasted_iota(jnp.int32, (1, NPAD), 1)
    s_m = jnp.where((prob_l > SCORE_T) & (lanes < NBOX), prob_l, NEG)
    z = jnp.zeros((3, NPAD), jnp.float32)
    out_ref[0] = jnp.concatenate([s_m, yy1, xx1, yy2, xx2, z], axis=0)


def _prep(rois_t, loc_t, sc_t):
    return pl.pallas_call(
        _prep_body,
        grid=(NOUT,),
        in_specs=[
            pl.BlockSpec((8, NPAD), lambda g: (0, 0)),
            pl.BlockSpec((1, 4, NPAD), lambda g: (g + 1, 0, 0)),
            pl.BlockSpec((24, NPAD), lambda g: (0, 0)),
        ],
        out_specs=pl.BlockSpec((1, 8, NPAD), lambda g: (g, 0, 0)),
        out_shape=jax.ShapeDtypeStruct((NOUT, 8, NPAD), jnp.float32),
    )(rois_t, loc_t, sc_t)


def _sc_nms(packed):
    mesh = plsc.VectorSubcoreMesh(
        core_axis_name="c", subcore_axis_name="s", num_cores=2,
        num_subcores=16)

    @functools.partial(
        pl.kernel,
        mesh=mesh,
        out_type=jax.ShapeDtypeStruct((NOUT * 5 * NPAD,), jnp.float32),
        compiler_params=pltpu.CompilerParams(needs_layout_passes=False),
        scratch_types=[pltpu.VMEM((CPAD,), jnp.float32)] * 5
        + [pltpu.VMEM((CPAD,), jnp.float32)] * 6
        + [pltpu.VMEM((CPAD,), jnp.int32)] * 2,
    )
    def k(in_hbm, out_hbm, s_v, y1_v, x1_v, y2_v, x2_v,
          cs, cy1, cx1, cy2, cx2, car, cidx, ci2):
        wid = lax.axis_index("s") * 2 + lax.axis_index("c")

        @pl.when(wid < NOUT)
        def _():
            c = wid
            ibase = c * 8 * NPAD
            pltpu.sync_copy(in_hbm.at[pl.ds(ibase + 0 * NPAD, NPAD)],
                            s_v.at[pl.ds(0, NPAD)])
            pltpu.sync_copy(in_hbm.at[pl.ds(ibase + 1 * NPAD, NPAD)],
                            y1_v.at[pl.ds(0, NPAD)])
            pltpu.sync_copy(in_hbm.at[pl.ds(ibase + 2 * NPAD, NPAD)],
                            x1_v.at[pl.ds(0, NPAD)])
            pltpu.sync_copy(in_hbm.at[pl.ds(ibase + 3 * NPAD, NPAD)],
                            y2_v.at[pl.ds(0, NPAD)])
            pltpu.sync_copy(in_hbm.at[pl.ds(ibase + 4 * NPAD, NPAD)],
                            x2_v.at[pl.ds(0, NPAD)])

            zero16 = jnp.zeros((16,), jnp.float32)
            negs16 = jnp.full((16,), NEG, jnp.float32)
            lane = lax.iota(jnp.int32, 16)
            mask0 = lane == 0
            lane0z = jnp.zeros((16,), jnp.int32)

            def cbody(i, off):
                sl = pl.ds(i * 16, 16)
                sv = s_v[sl]
                msk = sv > 0.0
                a = y1_v[sl]
                b = x1_v[sl]
                cc = y2_v[sl]
                dd = x2_v[sl]
                ar = jnp.maximum(cc - a, 0.0) * jnp.maximum(dd - b, 0.0)
                pos = plsc.cumsum(msk.astype(jnp.int32))
                dst = off + pos - 1
                plsc.store_scatter(cs, [dst], sv, mask=msk)
                plsc.store_scatter(cy1, [dst], a, mask=msk)
                plsc.store_scatter(cx1, [dst], b, mask=msk)
                plsc.store_scatter(cy2, [dst], cc, mask=msk)
                plsc.store_scatter(cx2, [dst], dd, mask=msk)
                plsc.store_scatter(car, [dst], ar, mask=msk)
                plsc.store_scatter(cidx, [dst], i * 16 + lane, mask=msk)
                return off + pos[15]

            moff = lax.fori_loop(0, NB, cbody, jnp.int32(0))
            cs[pl.ds(moff, 16)] = negs16
            cidx[pl.ds(moff, 16)] = lane0z + NPAD
            nbc = (moff + 15) // 16

            def zbody(i, carry):
                sl = pl.ds(i * 16, 16)
                ci2[sl] = i * 16 + lane
                s_v[sl] = zero16
                y1_v[sl] = zero16
                x1_v[sl] = zero16
                y2_v[sl] = zero16
                x2_v[sl] = zero16
                return carry

            lax.fori_loop(0, NB + 1, zbody, 0)

            ROUND = 96

            def round_body(r, mc):
                nbc = (mc + 15) // 16

                def am_body(i, carv):
                    rm, ri = carv
                    sv = cs[pl.ds(i * 16, 16)]
                    upd = sv > rm
                    return (jnp.where(upd, sv, rm),
                            jnp.where(upd, i * 16 + lane, ri))

                rm0, ri0 = lax.fori_loop(
                    0, nbc, am_body, (negs16, jnp.zeros((16,), jnp.int32)))
                g0 = jnp.max(rm0)
                i0 = jnp.min(jnp.where(rm0 == g0, ri0, jnp.int32(1 << 30)))
                i0 = jnp.where(g0 > 0.0, i0, jnp.int32(1 << 30))

                def obody(t, carv):
                    gm, gi = carv
                    alive = gm > 0.0
                    gic = jnp.minimum(gi, mc)
                    sel = pl.ds(gic, 16)

                    def spl(ref):
                        return ref[sel].at[lane0z].get(
                            mode="promise_in_bounds")

                    svalv = spl(cs)
                    by1v = spl(cy1)
                    bx1v = spl(cx1)
                    by2v = spl(cy2)
                    bx2v = spl(cx2)
                    bareav = spl(car)
                    oidxv = cidx[sel]
                    plsc.store_scatter(s_v, [oidxv], svalv, mask=mask0)
                    plsc.store_scatter(y1_v, [oidxv], by1v, mask=mask0)
                    plsc.store_scatter(x1_v, [oidxv], bx1v, mask=mask0)
                    plsc.store_scatter(y2_v, [oidxv], by2v, mask=mask0)
                    plsc.store_scatter(x2_v, [oidxv], bx2v, mask=mask0)
                    plsc.store_scatter(cs, [ci2[sel]], negs16, mask=mask0)
                    nbi = jnp.where(alive, nbc, 0)

                    def sup(i, carv2):
                        rm, ri = carv2
                        sl = pl.ds(i * 16, 16)
                        sv = cs[sl]
                        tly = jnp.maximum(by1v, cy1[sl])
                        tlx = jnp.maximum(bx1v, cx1[sl])
                        bry = jnp.minimum(by2v, cy2[sl])
                        brx = jnp.minimum(bx2v, cx2[sl])
                        hh = jnp.maximum(bry - tly, 0.0)
                        ww = jnp.maximum(brx - tlx, 0.0)
                        inter = hh * ww
                        iou = inter / (bareav + car[sl] - inter + 1e-8)
                        snew = jnp.where(iou > IOU_T, NEG, sv)
                        cs[sl] = snew
                        upd = snew > rm
                        return (jnp.where(upd, snew, rm),
                                jnp.where(upd, i * 16 + lane, ri))

                    rm, ri = lax.fori_loop(
                        0, nbi, sup, (negs16, jnp.zeros((16,), jnp.int32)))
                    gm2 = jnp.max(rm)
                    gi2 = jnp.min(jnp.where(rm == gm2, ri, jnp.int32(1 << 30)))
                    gi2 = jnp.where(gm2 > 0.0, gi2, jnp.int32(1 << 30))
                    gm2 = jnp.where(alive, gm2, gm)
                    gi2 = jnp.where(alive, gi2, gi)
                    return gm2, gi2

                lax.fori_loop(0, ROUND, obody, (g0, i0))

                def rbody(i, off):
                    sl = pl.ds(i * 16, 16)
                    sv = cs[sl]
                    msk = sv > 0.0
                    a = cy1[sl]
                    b = cx1[sl]
                    cc = cy2[sl]
                    dd = cx2[sl]
                    ar = car[sl]
                    ix = cidx[sl]
                    pos = plsc.cumsum(msk.astype(jnp.int32))
                    dst = off + pos - 1
                    plsc.store_scatter(cs, [dst], sv, mask=msk)
                    plsc.store_scatter(cy1, [dst], a, mask=msk)
                    plsc.store_scatter(cx1, [dst], b, mask=msk)
                    plsc.store_scatter(cy2, [dst], cc, mask=msk)
                    plsc.store_scatter(cx2, [dst], dd, mask=msk)
                    plsc.store_scatter(car, [dst], ar, mask=msk)
                    plsc.store_scatter(cidx, [dst], ix, mask=msk)
                    return off + pos[15]

                mc2 = lax.fori_loop(0, nbc, rbody, jnp.int32(0))
                cs[pl.ds(mc2, 16)] = negs16
                cidx[pl.ds(mc2, 16)] = lane0z + NPAD
                return mc2

            nrounds = (moff + ROUND - 1) // ROUND
            lax.fori_loop(0, nrounds, round_body, moff)

            obase = c * 5 * NPAD
            pltpu.sync_copy(y1_v.at[pl.ds(0, NPAD)],
                            out_hbm.at[pl.ds(obase + 0 * NPAD, NPAD)])
            pltpu.sync_copy(x1_v.at[pl.ds(0, NPAD)],
                            out_hbm.at[pl.ds(obase + 1 * NPAD, NPAD)])
            pltpu.sync_copy(y2_v.at[pl.ds(0, NPAD)],
                            out_hbm.at[pl.ds(obase + 2 * NPAD, NPAD)])
            pltpu.sync_copy(x2_v.at[pl.ds(0, NPAD)],
                            out_hbm.at[pl.ds(obase + 3 * NPAD, NPAD)])
            pltpu.sync_copy(s_v.at[pl.ds(0, NPAD)],
                            out_hbm.at[pl.ds(obase + 4 * NPAD, NPAD)])

    return k(packed)


def kernel(rois, roi_cls_loc, roi_scores):
    rois_t = jnp.zeros((8, NPAD), jnp.float32).at[:4, :NBOX].set(
        rois.astype(jnp.float32).T)
    loc_t = jnp.zeros((21, 4, NPAD), jnp.float32).at[:, :, :NBOX].set(
        roi_cls_loc.astype(jnp.float32).T.reshape(21, 4, NBOX))
    sc_t = jnp.full((24, NPAD), -1e30, jnp.float32).at[:21, :NBOX].set(
        roi_scores.astype(jnp.float32).T)
    packed = _prep(rois_t, loc_t, sc_t)
    out5 = _sc_nms(packed.reshape(-1)).reshape(NOUT, 5, NPAD)
    return out5.transpose(0, 2, 1)[:, :NBOX, :]


# speedup vs baseline: 168.7014x; 1.0426x over previous
import functools

import jax
import jax.numpy as jnp
from jax import lax
from jax.experimental import pallas as pl
from jax.experimental.pallas import tpu as pltpu
from jax.experimental.pallas import tpu_sc as plsc

NBOX = 5000
NPAD = 5008
NOUT = 20
NB = NPAD // 16
CPAD = NPAD + 16
NEG = -3.0e38
SCORE_T = 0.05
IOU_T = 0.3
IMG_H = 600.0
IMG_W = 800.0


def _prep_body(rois_ref, loc_ref, sc_ref, out_ref):
    y1r = rois_ref[0:1, :]
    x1r = rois_ref[1:2, :]
    y2r = rois_ref[2:3, :]
    x2r = rois_ref[3:4, :]
    src_h = y2r - y1r
    src_w = x2r - x1r
    src_cy = y1r + 0.5 * src_h
    src_cx = x1r + 0.5 * src_w
    dy = loc_ref[0, 0:1, :] * 0.1 + 0.0
    dx = loc_ref[0, 1:2, :] * 0.1 + 0.0
    dh = loc_ref[0, 2:3, :] * 0.2 + 0.0
    dw = loc_ref[0, 3:4, :] * 0.2 + 0.2
    cy = dy * src_h + src_cy
    cx = dx * src_w + src_cx
    hh = jnp.exp(dh) * src_h
    ww = jnp.exp(dw) * src_w
    yy1 = jnp.clip(cy - 0.5 * hh, 0.0, IMG_H)
    xx1 = jnp.clip(cx - 0.5 * ww, 0.0, IMG_W)
    yy2 = jnp.clip(cy + 0.5 * hh, 0.0, IMG_H)
    xx2 = jnp.clip(cx + 0.5 * ww, 0.0, IMG_W)
    sc = sc_ref[...]
    m = jnp.max(sc, axis=0, keepdims=True)
    e = jnp.exp(sc - m)
    denom = jnp.sum(e, axis=0, keepdims=True)
    probs = e / denom
    g = pl.program_id(0)
    onehot = (lax.broadcasted_iota(jnp.int32, (24, 1), 0) == (g + 1)).astype(
        jnp.float32)
    prob_l = jnp.sum(probs * onehot, axis=0, keepdims=True)
    lanes = lax.broadcasted_iota(jnp.int32, (1, NPAD), 1)
    s_m = jnp.where((prob_l > SCORE_T) & (lanes < NBOX), prob_l, NEG)
    z = jnp.zeros((3, NPAD), jnp.float32)
    out_ref[0] = jnp.concatenate([s_m, yy1, xx1, yy2, xx2, z], axis=0)


def _prep(rois_t, loc_t, sc_t):
    return pl.pallas_call(
        _prep_body,
        grid=(NOUT,),
        in_specs=[
            pl.BlockSpec((8, NPAD), lambda g: (0, 0)),
            pl.BlockSpec((1, 4, NPAD), lambda g: (g + 1, 0, 0)),
            pl.BlockSpec((24, NPAD), lambda g: (0, 0)),
        ],
        out_specs=pl.BlockSpec((1, 8, NPAD), lambda g: (g, 0, 0)),
        out_shape=jax.ShapeDtypeStruct((NOUT, 8, NPAD), jnp.float32),
    )(rois_t, loc_t, sc_t)


def _sc_nms(packed):
    mesh = plsc.VectorSubcoreMesh(
        core_axis_name="c", subcore_axis_name="s", num_cores=2,
        num_subcores=16)

    @functools.partial(
        pl.kernel,
        mesh=mesh,
        out_type=jax.ShapeDtypeStruct((NOUT * 5 * NPAD,), jnp.float32),
        compiler_params=pltpu.CompilerParams(needs_layout_passes=False),
        scratch_types=[pltpu.VMEM((CPAD,), jnp.float32)] * 5
        + [pltpu.VMEM((CPAD,), jnp.float32)] * 6
        + [pltpu.VMEM((CPAD,), jnp.int32)] * 2,
    )
    def k(in_hbm, out_hbm, s_v, y1_v, x1_v, y2_v, x2_v,
          cs, cy1, cx1, cy2, cx2, car, cidx, ci2):
        wid = lax.axis_index("s") * 2 + lax.axis_index("c")

        @pl.when(wid < NOUT)
        def _():
            c = wid
            ibase = c * 8 * NPAD
            pltpu.sync_copy(in_hbm.at[pl.ds(ibase + 0 * NPAD, NPAD)],
                            s_v.at[pl.ds(0, NPAD)])
            pltpu.sync_copy(in_hbm.at[pl.ds(ibase + 1 * NPAD, NPAD)],
                            y1_v.at[pl.ds(0, NPAD)])
            pltpu.sync_copy(in_hbm.at[pl.ds(ibase + 2 * NPAD, NPAD)],
                            x1_v.at[pl.ds(0, NPAD)])
            pltpu.sync_copy(in_hbm.at[pl.ds(ibase + 3 * NPAD, NPAD)],
                            y2_v.at[pl.ds(0, NPAD)])
            pltpu.sync_copy(in_hbm.at[pl.ds(ibase + 4 * NPAD, NPAD)],
                            x2_v.at[pl.ds(0, NPAD)])

            zero16 = jnp.zeros((16,), jnp.float32)
            negs16 = jnp.full((16,), NEG, jnp.float32)
            lane = lax.iota(jnp.int32, 16)
            mask0 = lane == 0
            lane0z = jnp.zeros((16,), jnp.int32)

            def cbody(i, off):
                sl = pl.ds(i * 16, 16)
                sv = s_v[sl]
                msk = sv > 0.0
                a = y1_v[sl]
                b = x1_v[sl]
                cc = y2_v[sl]
                dd = x2_v[sl]
                ar = jnp.maximum(cc - a, 0.0) * jnp.maximum(dd - b, 0.0)
                pos = plsc.cumsum(msk.astype(jnp.int32))
                dst = off + pos - 1
                plsc.store_scatter(cs, [dst], sv, mask=msk)
                plsc.store_scatter(cy1, [dst], a, mask=msk)
                plsc.store_scatter(cx1, [dst], b, mask=msk)
                plsc.store_scatter(cy2, [dst], cc, mask=msk)
                plsc.store_scatter(cx2, [dst], dd, mask=msk)
                plsc.store_scatter(car, [dst], ar, mask=msk)
                plsc.store_scatter(cidx, [dst], i * 16 + lane, mask=msk)
                return off + pos[15]

            moff = lax.fori_loop(0, NB, cbody, jnp.int32(0))
            cs[pl.ds(moff, 16)] = negs16
            cidx[pl.ds(moff, 16)] = lane0z + NPAD
            nbc = (moff + 15) // 16

            def zbody(i, carry):
                sl = pl.ds(i * 16, 16)
                ci2[sl] = i * 16 + lane
                s_v[sl] = zero16
                y1_v[sl] = zero16
                x1_v[sl] = zero16
                y2_v[sl] = zero16
                x2_v[sl] = zero16
                return carry

            lax.fori_loop(0, NB + 1, zbody, 0)

            ROUND = 64

            def round_body(r, mc):
                nbc = (mc + 15) // 16

                def am_body(i, carv):
                    rm, ri = carv
                    sv = cs[pl.ds(i * 16, 16)]
                    upd = sv > rm
                    return (jnp.where(upd, sv, rm),
                            jnp.where(upd, i * 16 + lane, ri))

                rm0, ri0 = lax.fori_loop(
                    0, nbc, am_body, (negs16, jnp.zeros((16,), jnp.int32)))
                g0 = jnp.max(rm0)
                i0 = jnp.min(jnp.where(rm0 == g0, ri0, jnp.int32(1 << 30)))
                i0 = jnp.where(g0 > 0.0, i0, jnp.int32(1 << 30))

                def obody(t, carv):
                    gm, gi = carv
                    alive = gm > 0.0
                    gic = jnp.minimum(gi, mc)
                    sel = pl.ds(gic, 16)

                    def spl(ref):
                        return ref[sel].at[lane0z].get(
                            mode="promise_in_bounds")

                    svalv = spl(cs)
                    by1v = spl(cy1)
                    bx1v = spl(cx1)
                    by2v = spl(cy2)
                    bx2v = spl(cx2)
                    bareav = spl(car)
                    oidxv = cidx[sel]
                    plsc.store_scatter(s_v, [oidxv], svalv, mask=mask0)
                    plsc.store_scatter(y1_v, [oidxv], by1v, mask=mask0)
                    plsc.store_scatter(x1_v, [oidxv], bx1v, mask=mask0)
                    plsc.store_scatter(y2_v, [oidxv], by2v, mask=mask0)
                    plsc.store_scatter(x2_v, [oidxv], bx2v, mask=mask0)
                    plsc.store_scatter(cs, [ci2[sel]], negs16, mask=mask0)
                    nbi = jnp.where(alive, nbc, 0)

                    def sup(i, carv2):
                        rm, ri = carv2
                        sl = pl.ds(i * 16, 16)
                        sv = cs[sl]
                        tly = jnp.maximum(by1v, cy1[sl])
                        tlx = jnp.maximum(bx1v, cx1[sl])
                        bry = jnp.minimum(by2v, cy2[sl])
                        brx = jnp.minimum(bx2v, cx2[sl])
                        hh = jnp.maximum(bry - tly, 0.0)
                        ww = jnp.maximum(brx - tlx, 0.0)
                        inter = hh * ww
                        iou = inter / (bareav + car[sl] - inter + 1e-8)
                        snew = jnp.where(iou > IOU_T, NEG, sv)
                        cs[sl] = snew
                        upd = snew > rm
                        return (jnp.where(upd, snew, rm),
                                jnp.where(upd, i * 16 + lane, ri))

                    rm, ri = lax.fori_loop(
                        0, nbi, sup, (negs16, jnp.zeros((16,), jnp.int32)))
                    gm2 = jnp.max(rm)
                    gi2 = jnp.min(jnp.where(rm == gm2, ri, jnp.int32(1 << 30)))
                    gi2 = jnp.where(gm2 > 0.0, gi2, jnp.int32(1 << 30))
                    gm2 = jnp.where(alive, gm2, gm)
                    gi2 = jnp.where(alive, gi2, gi)
                    return gm2, gi2

                lax.fori_loop(0, ROUND, obody, (g0, i0))

                def rbody(i, off):
                    sl = pl.ds(i * 16, 16)
                    sv = cs[sl]
                    msk = sv > 0.0
                    a = cy1[sl]
                    b = cx1[sl]
                    cc = cy2[sl]
                    dd = cx2[sl]
                    ar = car[sl]
                    ix = cidx[sl]
                    pos = plsc.cumsum(msk.astype(jnp.int32))
                    dst = off + pos - 1
                    plsc.store_scatter(cs, [dst], sv, mask=msk)
                    plsc.store_scatter(cy1, [dst], a, mask=msk)
                    plsc.store_scatter(cx1, [dst], b, mask=msk)
                    plsc.store_scatter(cy2, [dst], cc, mask=msk)
                    plsc.store_scatter(cx2, [dst], dd, mask=msk)
                    plsc.store_scatter(car, [dst], ar, mask=msk)
                    plsc.store_scatter(cidx, [dst], ix, mask=msk)
                    return off + pos[15]

                mc2 = lax.fori_loop(0, nbc, rbody, jnp.int32(0))
                cs[pl.ds(mc2, 16)] = negs16
                cidx[pl.ds(mc2, 16)] = lane0z + NPAD
                return mc2

            nrounds = (moff + ROUND - 1) // ROUND
            lax.fori_loop(0, nrounds, round_body, moff)

            obase = c * 5 * NPAD
            pltpu.sync_copy(y1_v.at[pl.ds(0, NPAD)],
                            out_hbm.at[pl.ds(obase + 0 * NPAD, NPAD)])
            pltpu.sync_copy(x1_v.at[pl.ds(0, NPAD)],
                            out_hbm.at[pl.ds(obase + 1 * NPAD, NPAD)])
            pltpu.sync_copy(y2_v.at[pl.ds(0, NPAD)],
                            out_hbm.at[pl.ds(obase + 2 * NPAD, NPAD)])
            pltpu.sync_copy(x2_v.at[pl.ds(0, NPAD)],
                            out_hbm.at[pl.ds(obase + 3 * NPAD, NPAD)])
            pltpu.sync_copy(s_v.at[pl.ds(0, NPAD)],
                            out_hbm.at[pl.ds(obase + 4 * NPAD, NPAD)])

    return k(packed)


def kernel(rois, roi_cls_loc, roi_scores):
    rois_t = jnp.zeros((8, NPAD), jnp.float32).at[:4, :NBOX].set(
        rois.astype(jnp.float32).T)
    loc_t = jnp.zeros((21, 4, NPAD), jnp.float32).at[:, :, :NBOX].set(
        roi_cls_loc.astype(jnp.float32).T.reshape(21, 4, NBOX))
    sc_t = jnp.full((24, NPAD), -1e30, jnp.float32).at[:21, :NBOX].set(
        roi_scores.astype(jnp.float32).T)
    packed = _prep(rois_t, loc_t, sc_t)
    out5 = _sc_nms(packed.reshape(-1)).reshape(NOUT, 5, NPAD)
    return out5.transpose(0, 2, 1)[:, :NBOX, :]


# speedup vs baseline: 174.7514x; 1.0359x over previous
import functools

import jax
import jax.numpy as jnp
from jax import lax
from jax.experimental import pallas as pl
from jax.experimental.pallas import tpu as pltpu
from jax.experimental.pallas import tpu_sc as plsc

NBOX = 5000
NPAD = 5008
NOUT = 20
NB = NPAD // 16
CPAD = NPAD + 16
NEG = -3.0e38
SCORE_T = 0.05
IOU_T = 0.3
IMG_H = 600.0
IMG_W = 800.0


def _prep_body(rois_ref, loc_ref, sc_ref, out_ref):
    y1r = rois_ref[0:1, :]
    x1r = rois_ref[1:2, :]
    y2r = rois_ref[2:3, :]
    x2r = rois_ref[3:4, :]
    src_h = y2r - y1r
    src_w = x2r - x1r
    src_cy = y1r + 0.5 * src_h
    src_cx = x1r + 0.5 * src_w
    dy = loc_ref[0, 0:1, :] * 0.1 + 0.0
    dx = loc_ref[0, 1:2, :] * 0.1 + 0.0
    dh = loc_ref[0, 2:3, :] * 0.2 + 0.0
    dw = loc_ref[0, 3:4, :] * 0.2 + 0.2
    cy = dy * src_h + src_cy
    cx = dx * src_w + src_cx
    hh = jnp.exp(dh) * src_h
    ww = jnp.exp(dw) * src_w
    yy1 = jnp.clip(cy - 0.5 * hh, 0.0, IMG_H)
    xx1 = jnp.clip(cx - 0.5 * ww, 0.0, IMG_W)
    yy2 = jnp.clip(cy + 0.5 * hh, 0.0, IMG_H)
    xx2 = jnp.clip(cx + 0.5 * ww, 0.0, IMG_W)
    sc = sc_ref[...]
    m = jnp.max(sc, axis=0, keepdims=True)
    e = jnp.exp(sc - m)
    denom = jnp.sum(e, axis=0, keepdims=True)
    probs = e / denom
    g = pl.program_id(0)
    onehot = (lax.broadcasted_iota(jnp.int32, (24, 1), 0) == (g + 1)).astype(
        jnp.float32)
    prob_l = jnp.sum(probs * onehot, axis=0, keepdims=True)
    lanes = lax.broadcasted_iota(jnp.int32, (1, NPAD), 1)
    s_m = jnp.where((prob_l > SCORE_T) & (lanes < NBOX), prob_l, NEG)
    z = jnp.zeros((3, NPAD), jnp.float32)
    out_ref[0] = jnp.concatenate([s_m, yy1, xx1, yy2, xx2, z], axis=0)


def _prep(rois_t, loc_t, sc_t):
    return pl.pallas_call(
        _prep_body,
        grid=(NOUT,),
        in_specs=[
            pl.BlockSpec((8, NPAD), lambda g: (0, 0)),
            pl.BlockSpec((1, 4, NPAD), lambda g: (g + 1, 0, 0)),
            pl.BlockSpec((24, NPAD), lambda g: (0, 0)),
        ],
        out_specs=pl.BlockSpec((1, 8, NPAD), lambda g: (g, 0, 0)),
        out_shape=jax.ShapeDtypeStruct((NOUT, 8, NPAD), jnp.float32),
    )(rois_t, loc_t, sc_t)


def _sc_nms(packed):
    mesh = plsc.VectorSubcoreMesh(
        core_axis_name="c", subcore_axis_name="s", num_cores=2,
        num_subcores=16)

    @functools.partial(
        pl.kernel,
        mesh=mesh,
        out_type=jax.ShapeDtypeStruct((NOUT * 5 * NPAD,), jnp.float32),
        compiler_params=pltpu.CompilerParams(needs_layout_passes=False),
        scratch_types=[pltpu.VMEM((CPAD,), jnp.float32)] * 5
        + [pltpu.VMEM((CPAD,), jnp.float32)] * 6
        + [pltpu.VMEM((CPAD,), jnp.int32)] * 2,
    )
    def k(in_hbm, out_hbm, s_v, y1_v, x1_v, y2_v, x2_v,
          cs, cy1, cx1, cy2, cx2, car, cidx, ci2):
        wid = lax.axis_index("s") * 2 + lax.axis_index("c")

        @pl.when(wid < NOUT)
        def _():
            c = wid
            ibase = c * 8 * NPAD
            pltpu.sync_copy(in_hbm.at[pl.ds(ibase + 0 * NPAD, NPAD)],
                            s_v.at[pl.ds(0, NPAD)])
            pltpu.sync_copy(in_hbm.at[pl.ds(ibase + 1 * NPAD, NPAD)],
                            y1_v.at[pl.ds(0, NPAD)])
            pltpu.sync_copy(in_hbm.at[pl.ds(ibase + 2 * NPAD, NPAD)],
                            x1_v.at[pl.ds(0, NPAD)])
            pltpu.sync_copy(in_hbm.at[pl.ds(ibase + 3 * NPAD, NPAD)],
                            y2_v.at[pl.ds(0, NPAD)])
            pltpu.sync_copy(in_hbm.at[pl.ds(ibase + 4 * NPAD, NPAD)],
                            x2_v.at[pl.ds(0, NPAD)])

            zero16 = jnp.zeros((16,), jnp.float32)
            negs16 = jnp.full((16,), NEG, jnp.float32)
            lane = lax.iota(jnp.int32, 16)
            mask0 = lane == 0
            lane0z = jnp.zeros((16,), jnp.int32)

            def cbody(i, off):
                sl = pl.ds(i * 16, 16)
                sv = s_v[sl]
                msk = sv > 0.0
                a = y1_v[sl]
                b = x1_v[sl]
                cc = y2_v[sl]
                dd = x2_v[sl]
                ar = jnp.maximum(cc - a, 0.0) * jnp.maximum(dd - b, 0.0)
                pos = plsc.cumsum(msk.astype(jnp.int32))
                dst = off + pos - 1
                plsc.store_scatter(cs, [dst], sv, mask=msk)
                plsc.store_scatter(cy1, [dst], a, mask=msk)
                plsc.store_scatter(cx1, [dst], b, mask=msk)
                plsc.store_scatter(cy2, [dst], cc, mask=msk)
                plsc.store_scatter(cx2, [dst], dd, mask=msk)
                plsc.store_scatter(car, [dst], ar, mask=msk)
                plsc.store_scatter(cidx, [dst], i * 16 + lane, mask=msk)
                return off + pos[15]

            moff = lax.fori_loop(0, NB, cbody, jnp.int32(0))
            cs[pl.ds(moff, 16)] = negs16
            cidx[pl.ds(moff, 16)] = lane0z + NPAD
            nbc = (moff + 15) // 16

            def zbody(i, carry):
                sl = pl.ds(i * 16, 16)
                ci2[sl] = i * 16 + lane
                s_v[sl] = zero16
                y1_v[sl] = zero16
                x1_v[sl] = zero16
                y2_v[sl] = zero16
                x2_v[sl] = zero16
                return carry

            lax.fori_loop(0, NB + 1, zbody, 0)

            ROUND = 40

            def round_body(r, mc):
                nbc = (mc + 15) // 16

                def am_body(i, carv):
                    rm, ri = carv
                    sv = cs[pl.ds(i * 16, 16)]
                    upd = sv > rm
                    return (jnp.where(upd, sv, rm),
                            jnp.where(upd, i * 16 + lane, ri))

                rm0, ri0 = lax.fori_loop(
                    0, nbc, am_body, (negs16, jnp.zeros((16,), jnp.int32)))
                g0 = jnp.max(rm0)
                i0 = jnp.min(jnp.where(rm0 == g0, ri0, jnp.int32(1 << 30)))
                i0 = jnp.where(g0 > 0.0, i0, jnp.int32(1 << 30))

                def obody(t, carv):
                    gm, gi = carv
                    alive = gm > 0.0
                    gic = jnp.minimum(gi, mc)
                    sel = pl.ds(gic, 16)

                    def spl(ref):
                        return ref[sel].at[lane0z].get(
                            mode="promise_in_bounds")

                    svalv = spl(cs)
                    by1v = spl(cy1)
                    bx1v = spl(cx1)
                    by2v = spl(cy2)
                    bx2v = spl(cx2)
                    bareav = spl(car)
                    oidxv = cidx[sel]
                    plsc.store_scatter(s_v, [oidxv], svalv, mask=mask0)
                    plsc.store_scatter(y1_v, [oidxv], by1v, mask=mask0)
                    plsc.store_scatter(x1_v, [oidxv], bx1v, mask=mask0)
                    plsc.store_scatter(y2_v, [oidxv], by2v, mask=mask0)
                    plsc.store_scatter(x2_v, [oidxv], bx2v, mask=mask0)
                    plsc.store_scatter(cs, [ci2[sel]], negs16, mask=mask0)
                    nbi = jnp.where(alive, nbc, 0)

                    def sup(i, carv2):
                        rm, ri = carv2
                        sl = pl.ds(i * 16, 16)
                        sv = cs[sl]
                        tly = jnp.maximum(by1v, cy1[sl])
                        tlx = jnp.maximum(bx1v, cx1[sl])
                        bry = jnp.minimum(by2v, cy2[sl])
                        brx = jnp.minimum(bx2v, cx2[sl])
                        hh = jnp.maximum(bry - tly, 0.0)
                        ww = jnp.maximum(brx - tlx, 0.0)
                        inter = hh * ww
                        iou = inter / (bareav + car[sl] - inter + 1e-8)
                        snew = jnp.where(iou > IOU_T, NEG, sv)
                        cs[sl] = snew
                        upd = snew > rm
                        return (jnp.where(upd, snew, rm),
                                jnp.where(upd, i * 16 + lane, ri))

                    rm, ri = lax.fori_loop(
                        0, nbi, sup, (negs16, jnp.zeros((16,), jnp.int32)))
                    gm2 = jnp.max(rm)
                    gi2 = jnp.min(jnp.where(rm == gm2, ri, jnp.int32(1 << 30)))
                    gi2 = jnp.where(gm2 > 0.0, gi2, jnp.int32(1 << 30))
                    gm2 = jnp.where(alive, gm2, gm)
                    gi2 = jnp.where(alive, gi2, gi)
                    return gm2, gi2

                lax.fori_loop(0, ROUND, obody, (g0, i0))

                def rbody(i, off):
                    sl = pl.ds(i * 16, 16)
                    sv = cs[sl]
                    msk = sv > 0.0
                    a = cy1[sl]
                    b = cx1[sl]
                    cc = cy2[sl]
                    dd = cx2[sl]
                    ar = car[sl]
                    ix = cidx[sl]
                    pos = plsc.cumsum(msk.astype(jnp.int32))
                    dst = off + pos - 1
                    plsc.store_scatter(cs, [dst], sv, mask=msk)
                    plsc.store_scatter(cy1, [dst], a, mask=msk)
                    plsc.store_scatter(cx1, [dst], b, mask=msk)
                    plsc.store_scatter(cy2, [dst], cc, mask=msk)
                    plsc.store_scatter(cx2, [dst], dd, mask=msk)
                    plsc.store_scatter(car, [dst], ar, mask=msk)
                    plsc.store_scatter(cidx, [dst], ix, mask=msk)
                    return off + pos[15]

                mc2 = lax.fori_loop(0, nbc, rbody, jnp.int32(0))
                cs[pl.ds(mc2, 16)] = negs16
                cidx[pl.ds(mc2, 16)] = lane0z + NPAD
                return mc2

            nrounds = (moff + ROUND - 1) // ROUND
            lax.fori_loop(0, nrounds, round_body, moff)

            obase = c * 5 * NPAD
            pltpu.sync_copy(y1_v.at[pl.ds(0, NPAD)],
                            out_hbm.at[pl.ds(obase + 0 * NPAD, NPAD)])
            pltpu.sync_copy(x1_v.at[pl.ds(0, NPAD)],
                            out_hbm.at[pl.ds(obase + 1 * NPAD, NPAD)])
            pltpu.sync_copy(y2_v.at[pl.ds(0, NPAD)],
                            out_hbm.at[pl.ds(obase + 2 * NPAD, NPAD)])
            pltpu.sync_copy(x2_v.at[pl.ds(0, NPAD)],
                            out_hbm.at[pl.ds(obase + 3 * NPAD, NPAD)])
            pltpu.sync_copy(s_v.at[pl.ds(0, NPAD)],
                            out_hbm.at[pl.ds(obase + 4 * NPAD, NPAD)])

    return k(packed)


def kernel(rois, roi_cls_loc, roi_scores):
    rois_t = jnp.zeros((8, NPAD), jnp.float32).at[:4, :NBOX].set(
        rois.astype(jnp.float32).T)
    loc_t = jnp.zeros((21, 4, NPAD), jnp.float32).at[:, :, :NBOX].set(
        roi_cls_loc.astype(jnp.float32).T.reshape(21, 4, NBOX))
    sc_t = jnp.full((24, NPAD), -1e30, jnp.float32).at[:21, :NBOX].set(
        roi_scores.astype(jnp.float32).T)
    packed = _prep(rois_t, loc_t, sc_t)
    out5 = _sc_nms(packed.reshape(-1)).reshape(NOUT, 5, NPAD)
    return out5.transpose(0, 2, 1)[:, :NBOX, :]


# speedup vs baseline: 177.7002x; 1.0169x over previous
import functools

import jax
import jax.numpy as jnp
from jax import lax
from jax.experimental import pallas as pl
from jax.experimental.pallas import tpu as pltpu
from jax.experimental.pallas import tpu_sc as plsc

NBOX = 5000
NPAD = 5008
NOUT = 20
NB = NPAD // 16
CPAD = NPAD + 16
NEG = -3.0e38
SCORE_T = 0.05
IOU_T = 0.3
IMG_H = 600.0
IMG_W = 800.0


def _prep_body(rois_ref, loc_ref, sc_ref, out_ref):
    y1r = rois_ref[0:1, :]
    x1r = rois_ref[1:2, :]
    y2r = rois_ref[2:3, :]
    x2r = rois_ref[3:4, :]
    src_h = y2r - y1r
    src_w = x2r - x1r
    src_cy = y1r + 0.5 * src_h
    src_cx = x1r + 0.5 * src_w
    dy = loc_ref[0, 0:1, :] * 0.1 + 0.0
    dx = loc_ref[0, 1:2, :] * 0.1 + 0.0
    dh = loc_ref[0, 2:3, :] * 0.2 + 0.0
    dw = loc_ref[0, 3:4, :] * 0.2 + 0.2
    cy = dy * src_h + src_cy
    cx = dx * src_w + src_cx
    hh = jnp.exp(dh) * src_h
    ww = jnp.exp(dw) * src_w
    yy1 = jnp.clip(cy - 0.5 * hh, 0.0, IMG_H)
    xx1 = jnp.clip(cx - 0.5 * ww, 0.0, IMG_W)
    yy2 = jnp.clip(cy + 0.5 * hh, 0.0, IMG_H)
    xx2 = jnp.clip(cx + 0.5 * ww, 0.0, IMG_W)
    sc = sc_ref[...]
    m = jnp.max(sc, axis=0, keepdims=True)
    e = jnp.exp(sc - m)
    denom = jnp.sum(e, axis=0, keepdims=True)
    probs = e / denom
    g = pl.program_id(0)
    onehot = (lax.broadcasted_iota(jnp.int32, (24, 1), 0) == (g + 1)).astype(
        jnp.float32)
    prob_l = jnp.sum(probs * onehot, axis=0, keepdims=True)
    lanes = lax.broadcasted_iota(jnp.int32, (1, NPAD), 1)
    s_m = jnp.where((prob_l > SCORE_T) & (lanes < NBOX), prob_l, NEG)
    z = jnp.zeros((3, NPAD), jnp.float32)
    out_ref[0] = jnp.concatenate([s_m, yy1, xx1, yy2, xx2, z], axis=0)


def _prep(rois_t, loc_t, sc_t):
    return pl.pallas_call(
        _prep_body,
        grid=(NOUT,),
        in_specs=[
            pl.BlockSpec((8, NPAD), lambda g: (0, 0)),
            pl.BlockSpec((1, 4, NPAD), lambda g: (g + 1, 0, 0)),
            pl.BlockSpec((24, NPAD), lambda g: (0, 0)),
        ],
        out_specs=pl.BlockSpec((1, 8, NPAD), lambda g: (g, 0, 0)),
        out_shape=jax.ShapeDtypeStruct((NOUT, 8, NPAD), jnp.float32),
    )(rois_t, loc_t, sc_t)


def _sc_nms(packed):
    mesh = plsc.VectorSubcoreMesh(
        core_axis_name="c", subcore_axis_name="s", num_cores=2,
        num_subcores=16)

    @functools.partial(
        pl.kernel,
        mesh=mesh,
        out_type=jax.ShapeDtypeStruct((NOUT * 5 * NPAD,), jnp.float32),
        compiler_params=pltpu.CompilerParams(needs_layout_passes=False),
        scratch_types=[pltpu.VMEM((CPAD,), jnp.float32)] * 5
        + [pltpu.VMEM((CPAD,), jnp.float32)] * 6
        + [pltpu.VMEM((CPAD,), jnp.int32)] * 2,
    )
    def k(in_hbm, out_hbm, s_v, y1_v, x1_v, y2_v, x2_v,
          cs, cy1, cx1, cy2, cx2, car, cidx, ci2):
        wid = lax.axis_index("s") * 2 + lax.axis_index("c")

        @pl.when(wid < NOUT)
        def _():
            c = wid
            ibase = c * 8 * NPAD
            pltpu.sync_copy(in_hbm.at[pl.ds(ibase + 0 * NPAD, NPAD)],
                            s_v.at[pl.ds(0, NPAD)])
            pltpu.sync_copy(in_hbm.at[pl.ds(ibase + 1 * NPAD, NPAD)],
                            y1_v.at[pl.ds(0, NPAD)])
            pltpu.sync_copy(in_hbm.at[pl.ds(ibase + 2 * NPAD, NPAD)],
                            x1_v.at[pl.ds(0, NPAD)])
            pltpu.sync_copy(in_hbm.at[pl.ds(ibase + 3 * NPAD, NPAD)],
                            y2_v.at[pl.ds(0, NPAD)])
            pltpu.sync_copy(in_hbm.at[pl.ds(ibase + 4 * NPAD, NPAD)],
                            x2_v.at[pl.ds(0, NPAD)])

            zero16 = jnp.zeros((16,), jnp.float32)
            negs16 = jnp.full((16,), NEG, jnp.float32)
            lane = lax.iota(jnp.int32, 16)
            mask0 = lane == 0
            lane0z = jnp.zeros((16,), jnp.int32)

            def cbody(i, off):
                sl = pl.ds(i * 16, 16)
                sv = s_v[sl]
                msk = sv > 0.0
                a = y1_v[sl]
                b = x1_v[sl]
                cc = y2_v[sl]
                dd = x2_v[sl]
                ar = jnp.maximum(cc - a, 0.0) * jnp.maximum(dd - b, 0.0)
                pos = plsc.cumsum(msk.astype(jnp.int32))
                dst = off + pos - 1
                plsc.store_scatter(cs, [dst], sv, mask=msk)
                plsc.store_scatter(cy1, [dst], a, mask=msk)
                plsc.store_scatter(cx1, [dst], b, mask=msk)
                plsc.store_scatter(cy2, [dst], cc, mask=msk)
                plsc.store_scatter(cx2, [dst], dd, mask=msk)
                plsc.store_scatter(car, [dst], ar, mask=msk)
                plsc.store_scatter(cidx, [dst], i * 16 + lane, mask=msk)
                return off + pos[15]

            moff = lax.fori_loop(0, NB, cbody, jnp.int32(0))
            cs[pl.ds(moff, 16)] = negs16
            cidx[pl.ds(moff, 16)] = lane0z + NPAD
            nbc = (moff + 15) // 16

            def zbody(i, carry):
                sl = pl.ds(i * 16, 16)
                ci2[sl] = i * 16 + lane
                s_v[sl] = zero16
                y1_v[sl] = zero16
                x1_v[sl] = zero16
                y2_v[sl] = zero16
                x2_v[sl] = zero16
                return carry

            lax.fori_loop(0, NB + 1, zbody, 0)

            ROUND = 24

            def round_body(r, mc):
                nbc = (mc + 15) // 16

                def am_body(i, carv):
                    rm, ri = carv
                    sv = cs[pl.ds(i * 16, 16)]
                    upd = sv > rm
                    return (jnp.where(upd, sv, rm),
                            jnp.where(upd, i * 16 + lane, ri))

                rm0, ri0 = lax.fori_loop(
                    0, nbc, am_body, (negs16, jnp.zeros((16,), jnp.int32)))
                g0 = jnp.max(rm0)
                i0 = jnp.min(jnp.where(rm0 == g0, ri0, jnp.int32(1 << 30)))
                i0 = jnp.where(g0 > 0.0, i0, jnp.int32(1 << 30))

                def obody(t, carv):
                    gm, gi = carv
                    alive = gm > 0.0
                    gic = jnp.minimum(gi, mc)
                    sel = pl.ds(gic, 16)

                    def spl(ref):
                        return ref[sel].at[lane0z].get(
                            mode="promise_in_bounds")

                    svalv = spl(cs)
                    by1v = spl(cy1)
                    bx1v = spl(cx1)
                    by2v = spl(cy2)
                    bx2v = spl(cx2)
                    bareav = spl(car)
                    oidxv = cidx[sel]
                    plsc.store_scatter(s_v, [oidxv], svalv, mask=mask0)
                    plsc.store_scatter(y1_v, [oidxv], by1v, mask=mask0)
                    plsc.store_scatter(x1_v, [oidxv], bx1v, mask=mask0)
                    plsc.store_scatter(y2_v, [oidxv], by2v, mask=mask0)
                    plsc.store_scatter(x2_v, [oidxv], bx2v, mask=mask0)
                    plsc.store_scatter(cs, [ci2[sel]], negs16, mask=mask0)
                    nbi = jnp.where(alive, nbc, 0)

                    def sup(i, carv2):
                        rm, ri = carv2
                        sl = pl.ds(i * 16, 16)
                        sv = cs[sl]
                        tly = jnp.maximum(by1v, cy1[sl])
                        tlx = jnp.maximum(bx1v, cx1[sl])
                        bry = jnp.minimum(by2v, cy2[sl])
                        brx = jnp.minimum(bx2v, cx2[sl])
                        hh = jnp.maximum(bry - tly, 0.0)
                        ww = jnp.maximum(brx - tlx, 0.0)
                        inter = hh * ww
                        iou = inter / (bareav + car[sl] - inter + 1e-8)
                        snew = jnp.where(iou > IOU_T, NEG, sv)
                        cs[sl] = snew
                        upd = snew > rm
                        return (jnp.where(upd, snew, rm),
                                jnp.where(upd, i * 16 + lane, ri))

                    rm, ri = lax.fori_loop(
                        0, nbi, sup, (negs16, jnp.zeros((16,), jnp.int32)))
                    gm2 = jnp.max(rm)
                    gi2 = jnp.min(jnp.where(rm == gm2, ri, jnp.int32(1 << 30)))
                    gi2 = jnp.where(gm2 > 0.0, gi2, jnp.int32(1 << 30))
                    gm2 = jnp.where(alive, gm2, gm)
                    gi2 = jnp.where(alive, gi2, gi)
                    return gm2, gi2

                lax.fori_loop(0, ROUND, obody, (g0, i0))

                def rbody(i, off):
                    sl = pl.ds(i * 16, 16)
                    sv = cs[sl]
                    msk = sv > 0.0
                    a = cy1[sl]
                    b = cx1[sl]
                    cc = cy2[sl]
                    dd = cx2[sl]
                    ar = car[sl]
                    ix = cidx[sl]
                    pos = plsc.cumsum(msk.astype(jnp.int32))
                    dst = off + pos - 1
                    plsc.store_scatter(cs, [dst], sv, mask=msk)
                    plsc.store_scatter(cy1, [dst], a, mask=msk)
                    plsc.store_scatter(cx1, [dst], b, mask=msk)
                    plsc.store_scatter(cy2, [dst], cc, mask=msk)
                    plsc.store_scatter(cx2, [dst], dd, mask=msk)
                    plsc.store_scatter(car, [dst], ar, mask=msk)
                    plsc.store_scatter(cidx, [dst], ix, mask=msk)
                    return off + pos[15]

                mc2 = lax.fori_loop(0, nbc, rbody, jnp.int32(0))
                cs[pl.ds(mc2, 16)] = negs16
                cidx[pl.ds(mc2, 16)] = lane0z + NPAD
                return mc2

            nrounds = (moff + ROUND - 1) // ROUND
            lax.fori_loop(0, nrounds, round_body, moff)

            obase = c * 5 * NPAD
            pltpu.sync_copy(y1_v.at[pl.ds(0, NPAD)],
                            out_hbm.at[pl.ds(obase + 0 * NPAD, NPAD)])
            pltpu.sync_copy(x1_v.at[pl.ds(0, NPAD)],
                            out_hbm.at[pl.ds(obase + 1 * NPAD, NPAD)])
            pltpu.sync_copy(y2_v.at[pl.ds(0, NPAD)],
                            out_hbm.at[pl.ds(obase + 2 * NPAD, NPAD)])
            pltpu.sync_copy(x2_v.at[pl.ds(0, NPAD)],
                            out_hbm.at[pl.ds(obase + 3 * NPAD, NPAD)])
            pltpu.sync_copy(s_v.at[pl.ds(0, NPAD)],
                            out_hbm.at[pl.ds(obase + 4 * NPAD, NPAD)])

    return k(packed)


def kernel(rois, roi_cls_loc, roi_scores):
    rois_t = jnp.zeros((8, NPAD), jnp.float32).at[:4, :NBOX].set(
        rois.astype(jnp.float32).T)
    loc_t = jnp.zeros((21, 4, NPAD), jnp.float32).at[:, :, :NBOX].set(
        roi_cls_loc.astype(jnp.float32).T.reshape(21, 4, NBOX))
    sc_t = jnp.full((24, NPAD), -1e30, jnp.float32).at[:21, :NBOX].set(
        roi_scores.astype(jnp.float32).T)
    packed = _prep(rois_t, loc_t, sc_t)
    out5 = _sc_nms(packed.reshape(-1)).reshape(NOUT, 5, NPAD)
    return out5.transpose(0, 2, 1)[:, :NBOX, :]


# speedup vs baseline: 355.0010x; 1.9978x over previous
import functools

import jax
import jax.numpy as jnp
from jax import lax
from jax.experimental import pallas as pl
from jax.experimental.pallas import tpu as pltpu
from jax.experimental.pallas import tpu_sc as plsc

NBOX = 5000
NPAD = 5008
NOUT = 20
NB = NPAD // 16
CPAD = NPAD + 16
NEG = -3.0e38
SCORE_T = 0.05
IOU_T = 0.3
IMG_H = 600.0
IMG_W = 800.0


def _prep_body(rois_ref, loc_ref, sc_ref, out_ref):
    y1r = rois_ref[0:1, :]
    x1r = rois_ref[1:2, :]
    y2r = rois_ref[2:3, :]
    x2r = rois_ref[3:4, :]
    src_h = y2r - y1r
    src_w = x2r - x1r
    src_cy = y1r + 0.5 * src_h
    src_cx = x1r + 0.5 * src_w
    dy = loc_ref[0, 0:1, :] * 0.1 + 0.0
    dx = loc_ref[0, 1:2, :] * 0.1 + 0.0
    dh = loc_ref[0, 2:3, :] * 0.2 + 0.0
    dw = loc_ref[0, 3:4, :] * 0.2 + 0.2
    cy = dy * src_h + src_cy
    cx = dx * src_w + src_cx
    hh = jnp.exp(dh) * src_h
    ww = jnp.exp(dw) * src_w
    yy1 = jnp.clip(cy - 0.5 * hh, 0.0, IMG_H)
    xx1 = jnp.clip(cx - 0.5 * ww, 0.0, IMG_W)
    yy2 = jnp.clip(cy + 0.5 * hh, 0.0, IMG_H)
    xx2 = jnp.clip(cx + 0.5 * ww, 0.0, IMG_W)
    sc = sc_ref[...]
    m = jnp.max(sc, axis=0, keepdims=True)
    e = jnp.exp(sc - m)
    denom = jnp.sum(e, axis=0, keepdims=True)
    probs = e / denom
    g = pl.program_id(0)
    onehot = (lax.broadcasted_iota(jnp.int32, (24, 1), 0) == (g + 1)).astype(
        jnp.float32)
    prob_l = jnp.sum(probs * onehot, axis=0, keepdims=True)
    lanes = lax.broadcasted_iota(jnp.int32, (1, NPAD), 1)
    s_m = jnp.where((prob_l > SCORE_T) & (lanes < NBOX), prob_l, NEG)
    z = jnp.zeros((3, NPAD), jnp.float32)
    out_ref[0] = jnp.concatenate([s_m, yy1, xx1, yy2, xx2, z], axis=0)


def _prep(rois_t, loc_t, sc_t):
    return pl.pallas_call(
        _prep_body,
        grid=(NOUT,),
        in_specs=[
            pl.BlockSpec((8, NPAD), lambda g: (0, 0)),
            pl.BlockSpec((1, 4, NPAD), lambda g: (g + 1, 0, 0)),
            pl.BlockSpec((24, NPAD), lambda g: (0, 0)),
        ],
        out_specs=pl.BlockSpec((1, 8, NPAD), lambda g: (g, 0, 0)),
        out_shape=jax.ShapeDtypeStruct((NOUT, 8, NPAD), jnp.float32),
    )(rois_t, loc_t, sc_t)


def _sc_nms(packed):
    mesh = plsc.VectorSubcoreMesh(
        core_axis_name="c", subcore_axis_name="s", num_cores=2,
        num_subcores=16)

    @functools.partial(
        pl.kernel,
        mesh=mesh,
        out_type=jax.ShapeDtypeStruct((NOUT * 5 * NPAD,), jnp.float32),
        compiler_params=pltpu.CompilerParams(needs_layout_passes=False),
        scratch_types=[pltpu.VMEM((CPAD,), jnp.float32)] * 5
        + [pltpu.VMEM((CPAD,), jnp.float32)] * 6
        + [pltpu.VMEM((CPAD,), jnp.int32)] * 2,
    )
    def k(in_hbm, out_hbm, s_v, y1_v, x1_v, y2_v, x2_v,
          cs, cy1, cx1, cy2, cx2, car, cidx, ci2):
        wid = lax.axis_index("s") * 2 + lax.axis_index("c")

        @pl.when(wid < NOUT)
        def _():
            c = wid
            ibase = c * 8 * NPAD
            pltpu.sync_copy(in_hbm.at[pl.ds(ibase + 0 * NPAD, NPAD)],
                            s_v.at[pl.ds(0, NPAD)])
            pltpu.sync_copy(in_hbm.at[pl.ds(ibase + 1 * NPAD, NPAD)],
                            y1_v.at[pl.ds(0, NPAD)])
            pltpu.sync_copy(in_hbm.at[pl.ds(ibase + 2 * NPAD, NPAD)],
                            x1_v.at[pl.ds(0, NPAD)])
            pltpu.sync_copy(in_hbm.at[pl.ds(ibase + 3 * NPAD, NPAD)],
                            y2_v.at[pl.ds(0, NPAD)])
            pltpu.sync_copy(in_hbm.at[pl.ds(ibase + 4 * NPAD, NPAD)],
                            x2_v.at[pl.ds(0, NPAD)])

            zero16 = jnp.zeros((16,), jnp.float32)
            negs16 = jnp.full((16,), NEG, jnp.float32)
            lane = lax.iota(jnp.int32, 16)
            mask0 = lane == 0
            lane0z = jnp.zeros((16,), jnp.int32)

            def cbody(i, off):
                sl = pl.ds(i * 16, 16)
                sv = s_v[sl]
                msk = sv > 0.0
                a = y1_v[sl]
                b = x1_v[sl]
                cc = y2_v[sl]
                dd = x2_v[sl]
                ar = jnp.maximum(cc - a, 0.0) * jnp.maximum(dd - b, 0.0)
                pos = plsc.cumsum(msk.astype(jnp.int32))
                dst = off + pos - 1
                plsc.store_scatter(cs, [dst], sv, mask=msk)
                plsc.store_scatter(cy1, [dst], a, mask=msk)
                plsc.store_scatter(cx1, [dst], b, mask=msk)
                plsc.store_scatter(cy2, [dst], cc, mask=msk)
                plsc.store_scatter(cx2, [dst], dd, mask=msk)
                plsc.store_scatter(car, [dst], ar, mask=msk)
                plsc.store_scatter(cidx, [dst], i * 16 + lane, mask=msk)
                return off + pos[15]

            moff = lax.fori_loop(0, NB, cbody, jnp.int32(0))
            cs[pl.ds(moff, 16)] = negs16
            cidx[pl.ds(moff, 16)] = lane0z + NPAD
            nbc = (moff + 15) // 16

            def zbody(i, carry):
                sl = pl.ds(i * 16, 16)
                ci2[sl] = i * 16 + lane
                s_v[sl] = zero16
                y1_v[sl] = zero16
                x1_v[sl] = zero16
                y2_v[sl] = zero16
                x2_v[sl] = zero16
                return carry

            lax.fori_loop(0, NB + 1, zbody, 0)

            ROUND = 24

            def round_body(r, mc):
                nbc = (mc + 15) // 16

                def am_body(i, carv):
                    rm, ri = carv
                    sv = cs[pl.ds(i * 16, 16)]
                    upd = sv > rm
                    return (jnp.where(upd, sv, rm),
                            jnp.where(upd, i * 16 + lane, ri))

                rm0, ri0 = lax.fori_loop(
                    0, nbc, am_body, (negs16, jnp.zeros((16,), jnp.int32)))
                g0 = jnp.max(rm0)
                i0 = jnp.min(jnp.where(rm0 == g0, ri0, jnp.int32(1 << 30)))
                i0 = jnp.where(g0 > 0.0, i0, jnp.int32(1 << 30))

                def obody(t, carv):
                    gm, gi = carv
                    alive = gm > 0.0
                    gic = jnp.minimum(gi, mc)
                    sel = pl.ds(gic, 16)

                    def spl(ref):
                        return ref[sel].at[lane0z].get(
                            mode="promise_in_bounds")

                    svalv = spl(cs)
                    by1v = spl(cy1)
                    bx1v = spl(cx1)
                    by2v = spl(cy2)
                    bx2v = spl(cx2)
                    bareav = spl(car)
                    oidxv = cidx[sel]
                    plsc.store_scatter(s_v, [oidxv], svalv, mask=mask0)
                    plsc.store_scatter(y1_v, [oidxv], by1v, mask=mask0)
                    plsc.store_scatter(x1_v, [oidxv], bx1v, mask=mask0)
                    plsc.store_scatter(y2_v, [oidxv], by2v, mask=mask0)
                    plsc.store_scatter(x2_v, [oidxv], bx2v, mask=mask0)
                    plsc.store_scatter(cs, [ci2[sel]], negs16, mask=mask0)
                    nbi = jnp.where(alive, nbc, 0)

                    @plsc.parallel_loop(
                        0, nbi, 1, unroll=4,
                        carry=(negs16, jnp.zeros((16,), jnp.int32)))
                    def sup_out(i, carv2):
                        rm, ri = carv2
                        sl = pl.ds(i * 16, 16)
                        sv = cs[sl]
                        tly = jnp.maximum(by1v, cy1[sl])
                        tlx = jnp.maximum(bx1v, cx1[sl])
                        bry = jnp.minimum(by2v, cy2[sl])
                        brx = jnp.minimum(bx2v, cx2[sl])
                        hh = jnp.maximum(bry - tly, 0.0)
                        ww = jnp.maximum(brx - tlx, 0.0)
                        inter = hh * ww
                        iou = inter / (bareav + car[sl] - inter + 1e-8)
                        snew = jnp.where(iou > IOU_T, NEG, sv)
                        cs[sl] = snew
                        upd = snew > rm
                        return (jnp.where(upd, snew, rm),
                                jnp.where(upd, i * 16 + lane, ri))

                    rm, ri = sup_out
                    gm2 = jnp.max(rm)
                    gi2 = jnp.min(jnp.where(rm == gm2, ri, jnp.int32(1 << 30)))
                    gi2 = jnp.where(gm2 > 0.0, gi2, jnp.int32(1 << 30))
                    gm2 = jnp.where(alive, gm2, gm)
                    gi2 = jnp.where(alive, gi2, gi)
                    return gm2, gi2

                lax.fori_loop(0, ROUND, obody, (g0, i0))

                def rbody(i, off):
                    sl = pl.ds(i * 16, 16)
                    sv = cs[sl]
                    msk = sv > 0.0
                    a = cy1[sl]
                    b = cx1[sl]
                    cc = cy2[sl]
                    dd = cx2[sl]
                    ar = car[sl]
                    ix = cidx[sl]
                    pos = plsc.cumsum(msk.astype(jnp.int32))
                    dst = off + pos - 1
                    plsc.store_scatter(cs, [dst], sv, mask=msk)
                    plsc.store_scatter(cy1, [dst], a, mask=msk)
                    plsc.store_scatter(cx1, [dst], b, mask=msk)
                    plsc.store_scatter(cy2, [dst], cc, mask=msk)
                    plsc.store_scatter(cx2, [dst], dd, mask=msk)
                    plsc.store_scatter(car, [dst], ar, mask=msk)
                    plsc.store_scatter(cidx, [dst], ix, mask=msk)
                    return off + pos[15]

                mc2 = lax.fori_loop(0, nbc, rbody, jnp.int32(0))
                cs[pl.ds(mc2, 16)] = negs16
                cidx[pl.ds(mc2, 16)] = lane0z + NPAD
                return mc2

            nrounds = (moff + ROUND - 1) // ROUND
            lax.fori_loop(0, nrounds, round_body, moff)

            obase = c * 5 * NPAD
            pltpu.sync_copy(y1_v.at[pl.ds(0, NPAD)],
                            out_hbm.at[pl.ds(obase + 0 * NPAD, NPAD)])
            pltpu.sync_copy(x1_v.at[pl.ds(0, NPAD)],
                            out_hbm.at[pl.ds(obase + 1 * NPAD, NPAD)])
            pltpu.sync_copy(y2_v.at[pl.ds(0, NPAD)],
                            out_hbm.at[pl.ds(obase + 2 * NPAD, NPAD)])
            pltpu.sync_copy(x2_v.at[pl.ds(0, NPAD)],
                            out_hbm.at[pl.ds(obase + 3 * NPAD, NPAD)])
            pltpu.sync_copy(s_v.at[pl.ds(0, NPAD)],
                            out_hbm.at[pl.ds(obase + 4 * NPAD, NPAD)])

    return k(packed)


def kernel(rois, roi_cls_loc, roi_scores):
    rois_t = jnp.zeros((8, NPAD), jnp.float32).at[:4, :NBOX].set(
        rois.astype(jnp.float32).T)
    loc_t = jnp.zeros((21, 4, NPAD), jnp.float32).at[:, :, :NBOX].set(
        roi_cls_loc.astype(jnp.float32).T.reshape(21, 4, NBOX))
    sc_t = jnp.full((24, NPAD), -1e30, jnp.float32).at[:21, :NBOX].set(
        roi_scores.astype(jnp.float32).T)
    packed = _prep(rois_t, loc_t, sc_t)
    out5 = _sc_nms(packed.reshape(-1)).reshape(NOUT, 5, NPAD)
    return out5.transpose(0, 2, 1)[:, :NBOX, :]


# speedup vs baseline: 364.8859x; 1.0278x over previous
import functools

import jax
import jax.numpy as jnp
from jax import lax
from jax.experimental import pallas as pl
from jax.experimental.pallas import tpu as pltpu
from jax.experimental.pallas import tpu_sc as plsc

NBOX = 5000
NPAD = 5008
NOUT = 20
NB = NPAD // 16
CPAD = NPAD + 16
NEG = -3.0e38
SCORE_T = 0.05
IOU_T = 0.3
IMG_H = 600.0
IMG_W = 800.0


def _prep_body(rois_ref, loc_ref, sc_ref, out_ref):
    y1r = rois_ref[0:1, :]
    x1r = rois_ref[1:2, :]
    y2r = rois_ref[2:3, :]
    x2r = rois_ref[3:4, :]
    src_h = y2r - y1r
    src_w = x2r - x1r
    src_cy = y1r + 0.5 * src_h
    src_cx = x1r + 0.5 * src_w
    dy = loc_ref[0, 0:1, :] * 0.1 + 0.0
    dx = loc_ref[0, 1:2, :] * 0.1 + 0.0
    dh = loc_ref[0, 2:3, :] * 0.2 + 0.0
    dw = loc_ref[0, 3:4, :] * 0.2 + 0.2
    cy = dy * src_h + src_cy
    cx = dx * src_w + src_cx
    hh = jnp.exp(dh) * src_h
    ww = jnp.exp(dw) * src_w
    yy1 = jnp.clip(cy - 0.5 * hh, 0.0, IMG_H)
    xx1 = jnp.clip(cx - 0.5 * ww, 0.0, IMG_W)
    yy2 = jnp.clip(cy + 0.5 * hh, 0.0, IMG_H)
    xx2 = jnp.clip(cx + 0.5 * ww, 0.0, IMG_W)
    sc = sc_ref[...]
    m = jnp.max(sc, axis=0, keepdims=True)
    e = jnp.exp(sc - m)
    denom = jnp.sum(e, axis=0, keepdims=True)
    probs = e / denom
    g = pl.program_id(0)
    onehot = (lax.broadcasted_iota(jnp.int32, (24, 1), 0) == (g + 1)).astype(
        jnp.float32)
    prob_l = jnp.sum(probs * onehot, axis=0, keepdims=True)
    lanes = lax.broadcasted_iota(jnp.int32, (1, NPAD), 1)
    s_m = jnp.where((prob_l > SCORE_T) & (lanes < NBOX), prob_l, NEG)
    z = jnp.zeros((3, NPAD), jnp.float32)
    out_ref[0] = jnp.concatenate([s_m, yy1, xx1, yy2, xx2, z], axis=0)


def _prep(rois_t, loc_t, sc_t):
    return pl.pallas_call(
        _prep_body,
        grid=(NOUT,),
        in_specs=[
            pl.BlockSpec((8, NPAD), lambda g: (0, 0)),
            pl.BlockSpec((1, 4, NPAD), lambda g: (g + 1, 0, 0)),
            pl.BlockSpec((24, NPAD), lambda g: (0, 0)),
        ],
        out_specs=pl.BlockSpec((1, 8, NPAD), lambda g: (g, 0, 0)),
        out_shape=jax.ShapeDtypeStruct((NOUT, 8, NPAD), jnp.float32),
    )(rois_t, loc_t, sc_t)


def _sc_nms(packed):
    mesh = plsc.VectorSubcoreMesh(
        core_axis_name="c", subcore_axis_name="s", num_cores=2,
        num_subcores=16)

    @functools.partial(
        pl.kernel,
        mesh=mesh,
        out_type=jax.ShapeDtypeStruct((NOUT * 5 * NPAD,), jnp.float32),
        compiler_params=pltpu.CompilerParams(needs_layout_passes=False),
        scratch_types=[pltpu.VMEM((CPAD,), jnp.float32)] * 5
        + [pltpu.VMEM((CPAD,), jnp.float32)] * 6
        + [pltpu.VMEM((CPAD,), jnp.int32)] * 2,
    )
    def k(in_hbm, out_hbm, s_v, y1_v, x1_v, y2_v, x2_v,
          cs, cy1, cx1, cy2, cx2, car, cidx, ci2):
        wid = lax.axis_index("s") * 2 + lax.axis_index("c")

        @pl.when(wid < NOUT)
        def _():
            c = wid
            ibase = c * 8 * NPAD
            pltpu.sync_copy(in_hbm.at[pl.ds(ibase + 0 * NPAD, NPAD)],
                            s_v.at[pl.ds(0, NPAD)])
            pltpu.sync_copy(in_hbm.at[pl.ds(ibase + 1 * NPAD, NPAD)],
                            y1_v.at[pl.ds(0, NPAD)])
            pltpu.sync_copy(in_hbm.at[pl.ds(ibase + 2 * NPAD, NPAD)],
                            x1_v.at[pl.ds(0, NPAD)])
            pltpu.sync_copy(in_hbm.at[pl.ds(ibase + 3 * NPAD, NPAD)],
                            y2_v.at[pl.ds(0, NPAD)])
            pltpu.sync_copy(in_hbm.at[pl.ds(ibase + 4 * NPAD, NPAD)],
                            x2_v.at[pl.ds(0, NPAD)])

            zero16 = jnp.zeros((16,), jnp.float32)
            negs16 = jnp.full((16,), NEG, jnp.float32)
            lane = lax.iota(jnp.int32, 16)
            mask0 = lane == 0
            lane0z = jnp.zeros((16,), jnp.int32)

            @plsc.parallel_loop(0, NB, 1, unroll=2, carry=jnp.int32(0))
            def moff(i, off):
                sl = pl.ds(i * 16, 16)
                sv = s_v[sl]
                msk = sv > 0.0
                a = y1_v[sl]
                b = x1_v[sl]
                cc = y2_v[sl]
                dd = x2_v[sl]
                ar = jnp.maximum(cc - a, 0.0) * jnp.maximum(dd - b, 0.0)
                pos = plsc.cumsum(msk.astype(jnp.int32))
                dst = off + pos - 1
                plsc.store_scatter(cs, [dst], sv, mask=msk)
                plsc.store_scatter(cy1, [dst], a, mask=msk)
                plsc.store_scatter(cx1, [dst], b, mask=msk)
                plsc.store_scatter(cy2, [dst], cc, mask=msk)
                plsc.store_scatter(cx2, [dst], dd, mask=msk)
                plsc.store_scatter(car, [dst], ar, mask=msk)
                plsc.store_scatter(cidx, [dst], i * 16 + lane, mask=msk)
                return off + pos[15]

            cs[pl.ds(moff, 16)] = negs16
            cidx[pl.ds(moff, 16)] = lane0z + NPAD
            nbc = (moff + 15) // 16

            @plsc.parallel_loop(0, NB + 1, 1, unroll=4)
            def _zero(i):
                sl = pl.ds(i * 16, 16)
                ci2[sl] = i * 16 + lane
                s_v[sl] = zero16
                y1_v[sl] = zero16
                x1_v[sl] = zero16
                y2_v[sl] = zero16
                x2_v[sl] = zero16

            ROUND = 24

            def round_body(r, mc):
                nbc = (mc + 15) // 16

                @plsc.parallel_loop(
                    0, nbc, 1, unroll=4,
                    carry=(negs16, jnp.zeros((16,), jnp.int32)))
                def am_out(i, carv):
                    rm, ri = carv
                    sv = cs[pl.ds(i * 16, 16)]
                    upd = sv > rm
                    return (jnp.where(upd, sv, rm),
                            jnp.where(upd, i * 16 + lane, ri))

                rm0, ri0 = am_out
                g0 = jnp.max(rm0)
                i0 = jnp.min(jnp.where(rm0 == g0, ri0, jnp.int32(1 << 30)))
                i0 = jnp.where(g0 > 0.0, i0, jnp.int32(1 << 30))

                def obody(t, carv):
                    gm, gi = carv
                    alive = gm > 0.0
                    gic = jnp.minimum(gi, mc)
                    sel = pl.ds(gic, 16)

                    def spl(ref):
                        return ref[sel].at[lane0z].get(
                            mode="promise_in_bounds")

                    svalv = spl(cs)
                    by1v = spl(cy1)
                    bx1v = spl(cx1)
                    by2v = spl(cy2)
                    bx2v = spl(cx2)
                    bareav = spl(car)
                    oidxv = cidx[sel]
                    plsc.store_scatter(s_v, [oidxv], svalv, mask=mask0)
                    plsc.store_scatter(y1_v, [oidxv], by1v, mask=mask0)
                    plsc.store_scatter(x1_v, [oidxv], bx1v, mask=mask0)
                    plsc.store_scatter(y2_v, [oidxv], by2v, mask=mask0)
                    plsc.store_scatter(x2_v, [oidxv], bx2v, mask=mask0)
                    plsc.store_scatter(cs, [ci2[sel]], negs16, mask=mask0)
                    nbi = jnp.where(alive, nbc, 0)

                    @plsc.parallel_loop(
                        0, nbi, 1, unroll=4,
                        carry=(negs16, jnp.zeros((16,), jnp.int32)))
                    def sup_out(i, carv2):
                        rm, ri = carv2
                        sl = pl.ds(i * 16, 16)
                        sv = cs[sl]
                        tly = jnp.maximum(by1v, cy1[sl])
                        tlx = jnp.maximum(bx1v, cx1[sl])
                        bry = jnp.minimum(by2v, cy2[sl])
                        brx = jnp.minimum(bx2v, cx2[sl])
                        hh = jnp.maximum(bry - tly, 0.0)
                        ww = jnp.maximum(brx - tlx, 0.0)
                        inter = hh * ww
                        iou = inter / (bareav + car[sl] - inter + 1e-8)
                        snew = jnp.where(iou > IOU_T, NEG, sv)
                        cs[sl] = snew
                        upd = snew > rm
                        return (jnp.where(upd, snew, rm),
                                jnp.where(upd, i * 16 + lane, ri))

                    rm, ri = sup_out
                    gm2 = jnp.max(rm)
                    gi2 = jnp.min(jnp.where(rm == gm2, ri, jnp.int32(1 << 30)))
                    gi2 = jnp.where(gm2 > 0.0, gi2, jnp.int32(1 << 30))
                    gm2 = jnp.where(alive, gm2, gm)
                    gi2 = jnp.where(alive, gi2, gi)
                    return gm2, gi2

                lax.fori_loop(0, ROUND, obody, (g0, i0))

                def rbody(i, off):
                    sl = pl.ds(i * 16, 16)
                    sv = cs[sl]
                    msk = sv > 0.0
                    a = cy1[sl]
                    b = cx1[sl]
                    cc = cy2[sl]
                    dd = cx2[sl]
                    ar = car[sl]
                    ix = cidx[sl]
                    pos = plsc.cumsum(msk.astype(jnp.int32))
                    dst = off + pos - 1
                    plsc.store_scatter(cs, [dst], sv, mask=msk)
                    plsc.store_scatter(cy1, [dst], a, mask=msk)
                    plsc.store_scatter(cx1, [dst], b, mask=msk)
                    plsc.store_scatter(cy2, [dst], cc, mask=msk)
                    plsc.store_scatter(cx2, [dst], dd, mask=msk)
                    plsc.store_scatter(car, [dst], ar, mask=msk)
                    plsc.store_scatter(cidx, [dst], ix, mask=msk)
                    return off + pos[15]

                mc2 = lax.fori_loop(0, nbc, rbody, jnp.int32(0))
                cs[pl.ds(mc2, 16)] = negs16
                cidx[pl.ds(mc2, 16)] = lane0z + NPAD
                return mc2

            nrounds = (moff + ROUND - 1) // ROUND
            lax.fori_loop(0, nrounds, round_body, moff)

            obase = c * 5 * NPAD
            pltpu.sync_copy(y1_v.at[pl.ds(0, NPAD)],
                            out_hbm.at[pl.ds(obase + 0 * NPAD, NPAD)])
            pltpu.sync_copy(x1_v.at[pl.ds(0, NPAD)],
                            out_hbm.at[pl.ds(obase + 1 * NPAD, NPAD)])
            pltpu.sync_copy(y2_v.at[pl.ds(0, NPAD)],
                            out_hbm.at[pl.ds(obase + 2 * NPAD, NPAD)])
            pltpu.sync_copy(x2_v.at[pl.ds(0, NPAD)],
                            out_hbm.at[pl.ds(obase + 3 * NPAD, NPAD)])
            pltpu.sync_copy(s_v.at[pl.ds(0, NPAD)],
                            out_hbm.at[pl.ds(obase + 4 * NPAD, NPAD)])

    return k(packed)


def kernel(rois, roi_cls_loc, roi_scores):
    rois_t = jnp.zeros((8, NPAD), jnp.float32).at[:4, :NBOX].set(
        rois.astype(jnp.float32).T)
    loc_t = jnp.zeros((21, 4, NPAD), jnp.float32).at[:, :, :NBOX].set(
        roi_cls_loc.astype(jnp.float32).T.reshape(21, 4, NBOX))
    sc_t = jnp.full((24, NPAD), -1e30, jnp.float32).at[:21, :NBOX].set(
        roi_scores.astype(jnp.float32).T)
    packed = _prep(rois_t, loc_t, sc_t)
    out5 = _sc_nms(packed.reshape(-1)).reshape(NOUT, 5, NPAD)
    return out5.transpose(0, 2, 1)[:, :NBOX, :]
